# baseline (speedup 1.0000x reference)
import functools

import jax
import jax.numpy as jnp
from jax import lax
from jax.experimental import pallas as pl
from jax.experimental.pallas import tpu as pltpu
from jax.experimental.pallas import tpu_sc as plsc

N_ROWS = 1000001
EMBED = 64
BATCH = 16384
CHUNK = 128
GROUP = 16


@functools.lru_cache(maxsize=None)
def _build_gather():
    info = plsc.get_sparse_core_info()
    nc, ns = info.num_cores, info.num_subcores
    nw = nc * ns
    b_per_w = BATCH // nw
    n_rounds = b_per_w // CHUNK
    mesh = plsc.VectorSubcoreMesh(core_axis_name="c", subcore_axis_name="s")

    @functools.partial(
        pl.kernel,
        mesh=mesh,
        compiler_params=pltpu.CompilerParams(needs_layout_passes=False),
        out_type=jax.ShapeDtypeStruct((BATCH, EMBED), jnp.float32),
        scratch_types=[
            pltpu.VMEM((n_rounds, CHUNK), jnp.int32),
            pltpu.VMEM((b_per_w,), jnp.int32),
            pltpu.VMEM((CHUNK, 2 * EMBED), jnp.float32),
            pltpu.VMEM((CHUNK, 2 * EMBED), jnp.float32),
            pltpu.VMEM((b_per_w, EMBED), jnp.float32),
            pltpu.SemaphoreType.DMA,
            pltpu.SemaphoreType.DMA,
        ],
    )
    def gather_kernel(table_hbm, pair_hbm, half_hbm, out_hbm,
                      pair_v, half_v, buf_a, buf_b, rows_v, sem_a, sem_b):
        wid = lax.axis_index("s") * nc + lax.axis_index("c")
        base = wid * b_per_w
        pltpu.sync_copy(pair_hbm.at[pl.ds(wid * n_rounds, n_rounds)], pair_v)
        pltpu.sync_copy(half_hbm.at[pl.ds(base, b_per_w)], half_v)

        bufs = (buf_a, buf_b)
        sems = (sem_a, sem_b)

        def fire(j):
            return pltpu.async_copy(
                table_hbm.at[pair_v.at[j]], bufs[j % 2], sems[j % 2])

        lane = lax.iota(jnp.int32, GROUP)

        def select(j, buf):
            def body_g(g, _):
                i_vec = g * GROUP + lane
                h64 = half_v[pl.ds(j * CHUNK + g * GROUP, GROUP)] * EMBED
                o_vec = j * CHUNK + i_vec

                def body_c(c, _):
                    c_vec = jnp.full((GROUP,), 0, jnp.int32) + c
                    x = plsc.load_gather(buf, [i_vec, h64 + c_vec])
                    plsc.store_scatter(rows_v, [o_vec, c_vec], x)
                    return _

                lax.fori_loop(0, EMBED, body_c, None)
                return _

            lax.fori_loop(0, CHUNK // GROUP, body_g, None)

        pending = fire(0)
        for j in range(n_rounds):
            pending.wait()
            if j + 1 < n_rounds:
                pending = fire(j + 1)
            select(j, bufs[j % 2])

        pltpu.sync_copy(rows_v, out_hbm.at[pl.ds(base, b_per_w)])

    return gather_kernel


def kernel(X, adj_list, ivectors, ovectors):
    xi = X.astype(jnp.int32)
    tbl128 = ivectors[: N_ROWS - 1].reshape((N_ROWS - 1) // 2, 2 * EMBED)
    pair2d = (xi >> 1).reshape(BATCH // CHUNK, CHUNK)
    return _build_gather()(tbl128, pair2d, xi & 1)


# speedup vs baseline: 1.8106x; 1.8106x over previous
import functools

import jax
import jax.numpy as jnp
from jax import lax
from jax.experimental import pallas as pl
from jax.experimental.pallas import tpu as pltpu
from jax.experimental.pallas import tpu_sc as plsc

N_ROWS = 1000001
EMBED = 64
BATCH = 16384
GROUP = 16


@functools.lru_cache(maxsize=None)
def _build_gather():
    info = plsc.get_sparse_core_info()
    nc, ns = info.num_cores, info.num_subcores
    nw = nc * ns
    b_per_w = BATCH // nw
    mesh = plsc.VectorSubcoreMesh(core_axis_name="c", subcore_axis_name="s")

    @functools.partial(
        pl.kernel,
        mesh=mesh,
        compiler_params=pltpu.CompilerParams(needs_layout_passes=False),
        out_type=jax.ShapeDtypeStruct((BATCH, EMBED), jnp.float32),
        scratch_types=[
            pltpu.VMEM((b_per_w,), jnp.int32),
            pltpu.VMEM((b_per_w, EMBED), jnp.float32),
            pltpu.SemaphoreType.DMA,
        ],
    )
    def gather_kernel(table_hbm, idx_hbm, out_hbm, idx_v, rows_v, sem):
        wid = lax.axis_index("s") * nc + lax.axis_index("c")
        base = wid * b_per_w
        pltpu.sync_copy(idx_hbm.at[pl.ds(base, b_per_w)], idx_v)
        lane = lax.iota(jnp.int32, GROUP)

        @plsc.parallel_loop(0, b_per_w, GROUP, unroll=4)
        def _(i):
            v = idx_v[pl.ds(i, GROUP)]
            for l in range(GROUP):
                r = jnp.sum(jnp.where(lane == l, v, 0))
                pltpu.async_copy(
                    table_hbm.at[pl.ds(r, 1)],
                    rows_v.at[pl.ds(i + l, 1)],
                    sem,
                )

        pltpu.make_async_copy(
            table_hbm.at[pl.ds(0, b_per_w)],
            rows_v,
            sem,
        ).wait()
        pltpu.sync_copy(rows_v, out_hbm.at[pl.ds(base, b_per_w)])

    return gather_kernel


def kernel(X, adj_list, ivectors, ovectors):
    return _build_gather()(ivectors, X.astype(jnp.int32))
